# baseline (speedup 1.0000x reference)
import functools

import jax
import jax.numpy as jnp
from jax import lax
from jax.experimental import pallas as pl
from jax.experimental.pallas import tpu as pltpu
from jax.experimental.pallas import tpu_sc as plsc

_N_ROWS = 4096
_N_COLS = 8192
_LANES = 16
_NW = 32
_ROWS_PER_W = _N_ROWS // _NW
_CHUNK_ROWS = 32
_SUBGROUPS = _CHUNK_ROWS // _LANES
_CHUNK_COLS = 512
_N_STAGES = _N_COLS // _CHUNK_COLS
_N_CHUNKS = _ROWS_PER_W // _CHUNK_ROWS
_TOT = _N_CHUNKS * _N_STAGES


def _sc_body(x_hbm, out_hbm, in_a, in_b, out_a, out_b, sem_ia, sem_ib,
             sem_oa, sem_ob):
    core = lax.axis_index("c")
    sub = lax.axis_index("s")
    wid = sub * 2 + core
    row_base = wid * _ROWS_PER_W

    row_iota = lax.iota(jnp.int32, _LANES)
    zeros = jnp.zeros((_LANES,), jnp.float32)
    group_rows = [row_iota + g * _LANES for g in range(_SUBGROUPS)]

    in_bufs = (in_a, in_b)
    out_bufs = (out_a, out_b)
    in_sems = (sem_ia, sem_ib)
    out_sems = (sem_oa, sem_ob)

    def hbm_slice(ref, s):
        r0 = row_base + (s // _N_STAGES) * _CHUNK_ROWS
        c0 = (s % _N_STAGES) * _CHUNK_COLS
        return ref.at[pl.ds(r0, _CHUNK_ROWS), pl.ds(c0, _CHUNK_COLS)]

    in_descs = [None] * _TOT
    out_descs = [None] * _TOT

    in_descs[0] = pltpu.async_copy(hbm_slice(x_hbm, 0), in_bufs[0],
                                   in_sems[0])
    accs = (zeros,) * _SUBGROUPS
    for s in range(_TOT):
        p = s % 2
        buf_in, buf_out = in_bufs[p], out_bufs[p]

        in_descs[s].wait()
        if s + 1 < _TOT:
            in_descs[s + 1] = pltpu.async_copy(
                hbm_slice(x_hbm, s + 1), in_bufs[1 - p], in_sems[1 - p])
        if s >= 2:
            out_descs[s - 2].wait()

        if s % _N_STAGES == 0:
            accs = (zeros,) * _SUBGROUPS

        col0 = jnp.zeros((_LANES,), jnp.int32)

        def step(j, carry):
            colv = carry[0]
            accs_in = carry[1:]
            new_accs = []
            for g in range(_SUBGROUPS):
                v = plsc.load_gather(buf_in, [group_rows[g], colv])
                a = accs_in[g] + v
                plsc.store_scatter(buf_out, [group_rows[g], colv], a)
                new_accs.append(a)
            return (colv + 1,) + tuple(new_accs)

        carry = lax.fori_loop(0, _CHUNK_COLS, step, (col0,) + accs)
        accs = carry[1:]

        out_descs[s] = pltpu.async_copy(buf_out, hbm_slice(out_hbm, s),
                                        out_sems[p])

    out_descs[_TOT - 2].wait()
    out_descs[_TOT - 1].wait()


@jax.jit
def kernel(x):
    mesh = plsc.VectorSubcoreMesh(core_axis_name="c", subcore_axis_name="s")
    buf = pltpu.VMEM((_CHUNK_ROWS, _CHUNK_COLS), jnp.float32)
    k = functools.partial(
        pl.kernel,
        mesh=mesh,
        out_type=jax.ShapeDtypeStruct((_N_ROWS, _N_COLS), jnp.float32),
        scratch_types=[buf, buf, buf, buf,
                       pltpu.SemaphoreType.DMA, pltpu.SemaphoreType.DMA,
                       pltpu.SemaphoreType.DMA, pltpu.SemaphoreType.DMA],
        compiler_params=pltpu.CompilerParams(
            use_tc_tiling_on_sc=False, needs_layout_passes=False),
    )(_sc_body)
    return k(x)


# speedup vs baseline: 17.8788x; 17.8788x over previous
import functools

import jax
import jax.numpy as jnp
from jax.experimental import pallas as pl
from jax.experimental.pallas import tpu as pltpu

_N_COLS = 8192
_LANES = 128
_GROUPS = _N_COLS // _LANES


def _cumsum_body(x_ref, o_ref, *, block_rows):
    li = jax.lax.broadcasted_iota(jnp.int32, (_LANES, _LANES), 0)
    lj = jax.lax.broadcasted_iota(jnp.int32, (_LANES, _LANES), 1)
    scan_mat = (li <= lj).astype(jnp.float32)

    carry = jnp.zeros((block_rows, 1), dtype=jnp.float32)
    for g in range(_GROUPS):
        xg = x_ref[:, g * _LANES:(g + 1) * _LANES]
        scan = jnp.dot(xg, scan_mat, preferred_element_type=jnp.float32)
        o_ref[:, g * _LANES:(g + 1) * _LANES] = scan + carry
        if g + 1 < _GROUPS:
            carry = carry + scan[:, _LANES - 1:_LANES]


@jax.jit
def kernel(x):
    n_rows, n_cols = x.shape
    block_rows = 256
    grid = (n_rows // block_rows,)
    return pl.pallas_call(
        functools.partial(_cumsum_body, block_rows=block_rows),
        grid=grid,
        in_specs=[pl.BlockSpec((block_rows, n_cols), lambda i: (i, 0))],
        out_specs=pl.BlockSpec((block_rows, n_cols), lambda i: (i, 0)),
        out_shape=jax.ShapeDtypeStruct((n_rows, n_cols), x.dtype),
    )(x)
